# baseline (speedup 1.0000x reference)
import jax
import jax.numpy as jnp
from jax.experimental import pallas as pl

T_MIN = 0.0
T_MAX = 2.0
MULTIPLIER = 20.0
MUL1 = 1.0 / MULTIPLIER
MUL2 = 1.0 / MULTIPLIER
B = 0.0

_CONST = B / MULTIPLIER + (T_MAX - T_MIN) + T_MIN
_BLOCK_CH = 4


def _ti_kernel(x1_ref, x2_ref, out_ref):
    d = (x1_ref[0] - x1_ref[1]) * MUL1 + (x2_ref[0] - x2_ref[1]) * MUL2
    out_ref[0] = jnp.minimum(_CONST + d, T_MAX)
    out_ref[1] = jnp.minimum(_CONST - d, T_MAX)


def kernel(tj1, tj2):
    _, ch, h, w = tj1.shape
    c = ch // 2
    cb = _BLOCK_CH
    grid = c // cb

    x1 = tj1.reshape(2, c, h, w)
    x2 = tj2.reshape(2, c, h, w)

    spec = pl.BlockSpec((2, cb, h, w), lambda i: (0, i, 0, 0))
    out = pl.pallas_call(
        _ti_kernel,
        grid=(grid,),
        in_specs=[spec, spec],
        out_specs=spec,
        out_shape=jax.ShapeDtypeStruct((2, c, h, w), jnp.float32),
    )(x1, x2)
    return out.reshape(ch, h, w)


# speedup vs baseline: 1.0586x; 1.0586x over previous
import jax
import jax.numpy as jnp
from jax.experimental import pallas as pl

T_MIN = 0.0
T_MAX = 2.0
MULTIPLIER = 20.0
MUL1 = 1.0 / MULTIPLIER
MUL2 = 1.0 / MULTIPLIER
B = 0.0

_CONST = B / MULTIPLIER + (T_MAX - T_MIN) + T_MIN
_BLOCK_CH = 12


def _ti_kernel(x1_ref, x2_ref, out_ref):
    d = (x1_ref[0] - x1_ref[1]) * MUL1 + (x2_ref[0] - x2_ref[1]) * MUL2
    out_ref[0] = jnp.minimum(_CONST + d, T_MAX)
    out_ref[1] = jnp.minimum(_CONST - d, T_MAX)


def kernel(tj1, tj2):
    _, ch, h, w = tj1.shape
    c = ch // 2
    cb = _BLOCK_CH
    grid = c // cb

    x1 = tj1.reshape(2, c, h, w)
    x2 = tj2.reshape(2, c, h, w)

    spec = pl.BlockSpec((2, cb, h, w), lambda i: (0, i, 0, 0))
    out = pl.pallas_call(
        _ti_kernel,
        grid=(grid,),
        in_specs=[spec, spec],
        out_specs=spec,
        out_shape=jax.ShapeDtypeStruct((2, c, h, w), jnp.float32),
    )(x1, x2)
    return out.reshape(ch, h, w)
